# baseline (speedup 1.0000x reference)
import functools

import jax
import jax.numpy as jnp
from jax import lax
from jax.experimental import pallas as pl
from jax.experimental.pallas import tpu as pltpu
import jax.experimental.pallas.tpu_sc as plsc

N = 10000
E = 320000
D = 128
H = 128
C = 64

NC = 2
NS = 16
NW = NC * NS
NP = 10240
RPT = NP // NS
EPW = E // NW
K = 80
ITERS = EPW // K
NBUF = 4
LOOPN = (ITERS - 1) // NBUF
DR = 8

_MESH = plsc.VectorSubcoreMesh(core_axis_name="c", subcore_axis_name="s",
                               num_cores=NC, num_subcores=NS)


def _sc_agg_body(h_hbm, src_hbm, dst_hbm, z2d_hbm, agg_out,
                 src_v, dst_v, sd_v, rows_v, shared_agg, isem, gsem):
    cid = lax.axis_index("c")
    sid = lax.axis_index("s")
    wid = cid * NS + sid
    ebase = wid * EPW
    sd_hbm = src_hbm.at[pl.ds(0, 2 * K)]

    pltpu.sync_copy(z2d_hbm, shared_agg.at[pl.ds(sid * RPT, RPT)])
    plsc.subcore_barrier()

    def idx_off(i):
        return pl.multiple_of(ebase + lax.rem(i, ITERS) * K, 8)

    for c in range(NBUF):
        pltpu.sync_copy(src_hbm.at[pl.ds(idx_off(c), K)], src_v[c])
        pltpu.sync_copy(dst_hbm.at[pl.ds(idx_off(c), K)], dst_v[c])
        pltpu.async_copy(h_hbm.at[src_v[c]], rows_v[c], gsem[c])

    def body(t, carry):
        i = NBUF * t
        for c in range(NBUF):
            pltpu.make_async_copy(h_hbm.at[src_v[c]], rows_v[c],
                                  gsem[c]).wait()
            pltpu.sync_copy(rows_v[c], shared_agg.at[dst_v[c]], add=True)
            pltpu.async_copy(src_hbm.at[pl.ds(idx_off(i + NBUF + c), K)],
                             src_v[c], isem[c])
            pltpu.async_copy(dst_hbm.at[pl.ds(idx_off(i + NBUF + c), K)],
                             dst_v[c], isem[c])
        for c in range(NBUF):
            pltpu.make_async_copy(sd_hbm, sd_v, isem[c]).wait()
            pltpu.async_copy(h_hbm.at[src_v[c]], rows_v[c], gsem[c])
        return carry

    lax.fori_loop(0, LOOPN, body, 0)
    pltpu.make_async_copy(h_hbm.at[src_v[0]], rows_v[0], gsem[0]).wait()
    pltpu.sync_copy(rows_v[0], shared_agg.at[dst_v[0]], add=True)
    for c in range(1, NBUF):
        pltpu.make_async_copy(h_hbm.at[src_v[c]], rows_v[c], gsem[c]).wait()
    plsc.subcore_barrier()

    pltpu.sync_copy(shared_agg.at[pl.ds(sid * RPT, RPT)],
                    agg_out.at[cid, pl.ds(sid * RPT, RPT)])


_sc_agg = pl.kernel(
    _sc_agg_body,
    out_type=[jax.ShapeDtypeStruct((NC, NP, H), jnp.float32)],
    mesh=_MESH,
    scratch_types=[
        [pltpu.VMEM((K,), jnp.int32)] * NBUF,
        [pltpu.VMEM((K,), jnp.int32)] * NBUF,
        pltpu.VMEM((2 * K,), jnp.int32),
        [pltpu.VMEM((K, H), jnp.float32)] * NBUF,
        pltpu.VMEM_SHARED((NP, H), jnp.float32),
        [pltpu.SemaphoreType.DMA] * NBUF,
        [pltpu.SemaphoreType.DMA] * NBUF,
    ],
    name="sc_sage_agg",
)


def _sc_deg_body(dst_hbm, zdeg_hbm, deg_out, dst_v, deg8_v):
    cid = lax.axis_index("c")
    sid = lax.axis_index("s")
    wid = cid * NS + sid

    pltpu.sync_copy(zdeg_hbm, deg8_v.at[pl.ds(0, DR * NP)])
    pltpu.sync_copy(dst_hbm.at[wid], dst_v)
    lanes = jax.lax.iota(jnp.int32, 16)
    row_base = jnp.bitwise_and(lanes, DR - 1) * NP
    mask_lo = lanes < 8
    mask_hi = lanes >= 8
    dump = DR * NP + lanes
    ones16 = jnp.ones((16,), jnp.float32)

    def body(g, carry):
        a16 = row_base + dst_v[pl.ds(g * 16, 16)]
        plsc.addupdate_scatter(deg8_v, [jnp.where(mask_lo, a16, dump)],
                               ones16)
        plsc.addupdate_scatter(deg8_v, [jnp.where(mask_hi, a16, dump)],
                               ones16)
        return carry

    lax.fori_loop(0, EPW // 16, body, 0)

    def dbody(g, carry):
        acc = deg8_v[pl.ds(g * 16, 16)]
        for r in range(1, DR):
            acc = acc + deg8_v[pl.ds(r * NP + g * 16, 16)]
        deg8_v[pl.ds(g * 16, 16)] = acc
        return carry

    lax.fori_loop(0, NP // 16, dbody, 0)
    pltpu.sync_copy(deg8_v.at[pl.ds(0, NP)], deg_out.at[wid])


_sc_deg = pl.kernel(
    _sc_deg_body,
    out_type=[jax.ShapeDtypeStruct((NW, NP), jnp.float32)],
    mesh=_MESH,
    scratch_types=[
        pltpu.VMEM((EPW,), jnp.int32),
        pltpu.VMEM((DR * NP + 16,), jnp.float32),
    ],
    compiler_params=pltpu.CompilerParams(needs_layout_passes=False),
    name="sc_sage_deg",
)


def _layer0_kernel(h_ref, agg_ref, parts_ref, ws_ref, wn_ref, b_ref,
                   out_ref, rdeg_ref):
    rdeg = (1.0 / jnp.maximum(jnp.sum(parts_ref[...], axis=0), 1.0))[:, None]
    rdeg_ref[...] = rdeg
    hn = (agg_ref[0] + agg_ref[1]) * rdeg
    z = (jnp.dot(h_ref[...], ws_ref[...], preferred_element_type=jnp.float32)
         + jnp.dot(hn, wn_ref[...], preferred_element_type=jnp.float32)
         + b_ref[...])
    out_ref[...] = jnp.maximum(z, 0.0)


def _layer_kernel(mode, h_ref, agg_ref, rdeg_ref, ws_ref, wn_ref, b_ref,
                  out_ref):
    hn = (agg_ref[0] + agg_ref[1]) * rdeg_ref[...]
    z = (jnp.dot(h_ref[...], ws_ref[...], preferred_element_type=jnp.float32)
         + jnp.dot(hn, wn_ref[...], preferred_element_type=jnp.float32)
         + b_ref[...])
    if mode == "relu":
        z = jnp.maximum(z, 0.0)
    elif mode == "logsoftmax":
        m = jnp.max(z, axis=-1, keepdims=True)
        e = z - m
        z = e - jnp.log(jnp.sum(jnp.exp(e), axis=-1, keepdims=True))
    out_ref[...] = z


RB = 1280
_GRID = NP // RB


def _tc_layer(h, agg, rdeg, ws, wn, b, mode):
    din, dout = ws.shape
    return pl.pallas_call(
        functools.partial(_layer_kernel, mode),
        grid=(_GRID,),
        in_specs=[
            pl.BlockSpec((RB, din), lambda i: (i, 0)),
            pl.BlockSpec((NC, RB, H), lambda i: (0, i, 0)),
            pl.BlockSpec((RB, 1), lambda i: (i, 0)),
            pl.BlockSpec((din, dout), lambda i: (0, 0)),
            pl.BlockSpec((H, dout), lambda i: (0, 0)),
            pl.BlockSpec((1, dout), lambda i: (0, 0)),
        ],
        out_specs=pl.BlockSpec((RB, dout), lambda i: (i, 0)),
        out_shape=jax.ShapeDtypeStruct((NP, dout), jnp.float32),
        name=f"tc_sage_layer_{mode}",
    )(h, agg, rdeg, ws, wn, b)


def _tc_layer0(h, agg, deg_parts, ws, wn, b):
    din, dout = ws.shape
    return pl.pallas_call(
        _layer0_kernel,
        grid=(_GRID,),
        in_specs=[
            pl.BlockSpec((RB, din), lambda i: (i, 0)),
            pl.BlockSpec((NC, RB, H), lambda i: (0, i, 0)),
            pl.BlockSpec((NW, RB), lambda i: (0, i)),
            pl.BlockSpec((din, dout), lambda i: (0, 0)),
            pl.BlockSpec((H, dout), lambda i: (0, 0)),
            pl.BlockSpec((1, dout), lambda i: (0, 0)),
        ],
        out_specs=[
            pl.BlockSpec((RB, dout), lambda i: (i, 0)),
            pl.BlockSpec((RB, 1), lambda i: (i, 0)),
        ],
        out_shape=[
            jax.ShapeDtypeStruct((NP, dout), jnp.float32),
            jax.ShapeDtypeStruct((NP, 1), jnp.float32),
        ],
        name="tc_sage_layer0",
    )(h, agg, deg_parts, ws, wn, b)


def kernel(x, edge_index, W_self0, W_neigh0, b0, W_self1, W_neigh1, b1,
           W_self2, W_neigh2, b2):
    src = edge_index[0]
    dst = edge_index[1]
    dst2 = dst.reshape(NW, EPW)
    x_pad = jnp.zeros((NP, D), jnp.float32).at[:N].set(x)
    z_h = jnp.zeros((RPT, H), jnp.float32)
    z_deg = jnp.zeros((DR * NP,), jnp.float32)

    (deg_parts,) = _sc_deg(dst2, z_deg)
    (agg0,) = _sc_agg(x_pad, src, dst, z_h)
    h1, rdeg = _tc_layer0(x_pad, agg0, deg_parts, W_self0, W_neigh0,
                          b0[None, :])
    (agg1,) = _sc_agg(h1, src, dst, z_h)
    h2 = _tc_layer(h1, agg1, rdeg, W_self1, W_neigh1, b1[None, :], "relu")
    (agg2,) = _sc_agg(h2, src, dst, z_h)
    out = _tc_layer(h2, agg2, rdeg, W_self2, W_neigh2, b2[None, :],
                    "logsoftmax")
    return out[:N]


# speedup vs baseline: 1.0645x; 1.0645x over previous
import functools

import jax
import jax.numpy as jnp
from jax import lax
from jax.experimental import pallas as pl
from jax.experimental.pallas import tpu as pltpu
import jax.experimental.pallas.tpu_sc as plsc

N = 10000
E = 320000
D = 128
H = 128
C = 64

NC = 2
NS = 16
NW = NC * NS
NP = 10240
RPT = NP // NS
EPW = E // NW
K = 128
ITERS = EPW // K
KT = EPW - ITERS * K
NBUF = 2
LOOPN = ITERS // NBUF
DR = 8

_MESH = plsc.VectorSubcoreMesh(core_axis_name="c", subcore_axis_name="s",
                               num_cores=NC, num_subcores=NS)


def _sc_agg_body(h_hbm, src_hbm, dst_hbm, z2d_hbm, agg_out,
                 src_v, dst_v, sd_v, rows_v, srct_v, dstt_v, rowst_v,
                 shared_agg, isem, gsem):
    cid = lax.axis_index("c")
    sid = lax.axis_index("s")
    wid = cid * NS + sid
    ebase = wid * EPW
    sd_hbm = src_hbm.at[pl.ds(0, 2 * K)]

    pltpu.sync_copy(z2d_hbm, shared_agg.at[pl.ds(sid * RPT, RPT)])
    plsc.subcore_barrier()

    def idx_off(i):
        return pl.multiple_of(ebase + lax.rem(i, ITERS) * K, 8)

    for c in range(NBUF):
        pltpu.sync_copy(src_hbm.at[pl.ds(idx_off(c), K)], src_v[c])
        pltpu.sync_copy(dst_hbm.at[pl.ds(idx_off(c), K)], dst_v[c])
        pltpu.async_copy(h_hbm.at[src_v[c]], rows_v[c], gsem[c])

    def body(t, carry):
        i = NBUF * t
        for c in range(NBUF):
            pltpu.make_async_copy(h_hbm.at[src_v[c]], rows_v[c],
                                  gsem[c]).wait()
            pltpu.sync_copy(rows_v[c], shared_agg.at[dst_v[c]], add=True)
            pltpu.async_copy(src_hbm.at[pl.ds(idx_off(i + NBUF + c), K)],
                             src_v[c], isem[c])
            pltpu.async_copy(dst_hbm.at[pl.ds(idx_off(i + NBUF + c), K)],
                             dst_v[c], isem[c])
        for c in range(NBUF):
            pltpu.make_async_copy(sd_hbm, sd_v, isem[c]).wait()
            pltpu.async_copy(h_hbm.at[src_v[c]], rows_v[c], gsem[c])
        return carry

    lax.fori_loop(0, LOOPN, body, 0)
    for c in range(NBUF):
        pltpu.make_async_copy(h_hbm.at[src_v[c]], rows_v[c], gsem[c]).wait()
    toff = pl.multiple_of(ebase + ITERS * K, 8)
    pltpu.sync_copy(src_hbm.at[pl.ds(toff, KT)], srct_v)
    pltpu.sync_copy(dst_hbm.at[pl.ds(toff, KT)], dstt_v)
    pltpu.async_copy(h_hbm.at[srct_v], rowst_v, gsem[0]).wait()
    pltpu.sync_copy(rowst_v, shared_agg.at[dstt_v], add=True)
    plsc.subcore_barrier()

    pltpu.sync_copy(shared_agg.at[pl.ds(sid * RPT, RPT)],
                    agg_out.at[cid, pl.ds(sid * RPT, RPT)])


_sc_agg = pl.kernel(
    _sc_agg_body,
    out_type=[jax.ShapeDtypeStruct((NC, NP, H), jnp.float32)],
    mesh=_MESH,
    scratch_types=[
        [pltpu.VMEM((K,), jnp.int32)] * NBUF,
        [pltpu.VMEM((K,), jnp.int32)] * NBUF,
        pltpu.VMEM((2 * K,), jnp.int32),
        [pltpu.VMEM((K, H), jnp.float32)] * NBUF,
        pltpu.VMEM((KT,), jnp.int32),
        pltpu.VMEM((KT,), jnp.int32),
        pltpu.VMEM((KT, H), jnp.float32),
        pltpu.VMEM_SHARED((NP, H), jnp.float32),
        [pltpu.SemaphoreType.DMA] * NBUF,
        [pltpu.SemaphoreType.DMA] * NBUF,
    ],
    name="sc_sage_agg",
)


def _sc_deg_body(dst_hbm, zdeg_hbm, deg_out, dst_v, deg8_v):
    cid = lax.axis_index("c")
    sid = lax.axis_index("s")
    wid = cid * NS + sid

    pltpu.sync_copy(zdeg_hbm, deg8_v.at[pl.ds(0, DR * NP)])
    pltpu.sync_copy(dst_hbm.at[wid], dst_v)
    lanes = jax.lax.iota(jnp.int32, 16)
    row_base = jnp.bitwise_and(lanes, DR - 1) * NP
    mask_lo = lanes < 8
    mask_hi = lanes >= 8
    dump = DR * NP + lanes
    ones16 = jnp.ones((16,), jnp.float32)

    def body(g, carry):
        a16 = row_base + dst_v[pl.ds(g * 16, 16)]
        plsc.addupdate_scatter(deg8_v, [jnp.where(mask_lo, a16, dump)],
                               ones16)
        plsc.addupdate_scatter(deg8_v, [jnp.where(mask_hi, a16, dump)],
                               ones16)
        return carry

    lax.fori_loop(0, EPW // 16, body, 0)

    def dbody(g, carry):
        acc = deg8_v[pl.ds(g * 16, 16)]
        for r in range(1, DR):
            acc = acc + deg8_v[pl.ds(r * NP + g * 16, 16)]
        deg8_v[pl.ds(g * 16, 16)] = acc
        return carry

    lax.fori_loop(0, NP // 16, dbody, 0)
    pltpu.sync_copy(deg8_v.at[pl.ds(0, NP)], deg_out.at[wid])


_sc_deg = pl.kernel(
    _sc_deg_body,
    out_type=[jax.ShapeDtypeStruct((NW, NP), jnp.float32)],
    mesh=_MESH,
    scratch_types=[
        pltpu.VMEM((EPW,), jnp.int32),
        pltpu.VMEM((DR * NP + 16,), jnp.float32),
    ],
    compiler_params=pltpu.CompilerParams(needs_layout_passes=False),
    name="sc_sage_deg",
)


def _layer0_kernel(h_ref, agg_ref, parts_ref, ws_ref, wn_ref, b_ref,
                   out_ref, rdeg_ref):
    rdeg = (1.0 / jnp.maximum(jnp.sum(parts_ref[...], axis=0), 1.0))[:, None]
    rdeg_ref[...] = rdeg
    hn = (agg_ref[0] + agg_ref[1]) * rdeg
    z = (jnp.dot(h_ref[...], ws_ref[...], preferred_element_type=jnp.float32)
         + jnp.dot(hn, wn_ref[...], preferred_element_type=jnp.float32)
         + b_ref[...])
    out_ref[...] = jnp.maximum(z, 0.0)


def _layer_kernel(mode, h_ref, agg_ref, rdeg_ref, ws_ref, wn_ref, b_ref,
                  out_ref):
    hn = (agg_ref[0] + agg_ref[1]) * rdeg_ref[...]
    z = (jnp.dot(h_ref[...], ws_ref[...], preferred_element_type=jnp.float32)
         + jnp.dot(hn, wn_ref[...], preferred_element_type=jnp.float32)
         + b_ref[...])
    if mode == "relu":
        z = jnp.maximum(z, 0.0)
    elif mode == "logsoftmax":
        m = jnp.max(z, axis=-1, keepdims=True)
        e = z - m
        z = e - jnp.log(jnp.sum(jnp.exp(e), axis=-1, keepdims=True))
    out_ref[...] = z


RB = 1280
_GRID = NP // RB


def _tc_layer(h, agg, rdeg, ws, wn, b, mode):
    din, dout = ws.shape
    return pl.pallas_call(
        functools.partial(_layer_kernel, mode),
        grid=(_GRID,),
        in_specs=[
            pl.BlockSpec((RB, din), lambda i: (i, 0)),
            pl.BlockSpec((NC, RB, H), lambda i: (0, i, 0)),
            pl.BlockSpec((RB, 1), lambda i: (i, 0)),
            pl.BlockSpec((din, dout), lambda i: (0, 0)),
            pl.BlockSpec((H, dout), lambda i: (0, 0)),
            pl.BlockSpec((1, dout), lambda i: (0, 0)),
        ],
        out_specs=pl.BlockSpec((RB, dout), lambda i: (i, 0)),
        out_shape=jax.ShapeDtypeStruct((NP, dout), jnp.float32),
        name=f"tc_sage_layer_{mode}",
    )(h, agg, rdeg, ws, wn, b)


def _tc_layer0(h, agg, deg_parts, ws, wn, b):
    din, dout = ws.shape
    return pl.pallas_call(
        _layer0_kernel,
        grid=(_GRID,),
        in_specs=[
            pl.BlockSpec((RB, din), lambda i: (i, 0)),
            pl.BlockSpec((NC, RB, H), lambda i: (0, i, 0)),
            pl.BlockSpec((NW, RB), lambda i: (0, i)),
            pl.BlockSpec((din, dout), lambda i: (0, 0)),
            pl.BlockSpec((H, dout), lambda i: (0, 0)),
            pl.BlockSpec((1, dout), lambda i: (0, 0)),
        ],
        out_specs=[
            pl.BlockSpec((RB, dout), lambda i: (i, 0)),
            pl.BlockSpec((RB, 1), lambda i: (i, 0)),
        ],
        out_shape=[
            jax.ShapeDtypeStruct((NP, dout), jnp.float32),
            jax.ShapeDtypeStruct((NP, 1), jnp.float32),
        ],
        name="tc_sage_layer0",
    )(h, agg, deg_parts, ws, wn, b)


def kernel(x, edge_index, W_self0, W_neigh0, b0, W_self1, W_neigh1, b1,
           W_self2, W_neigh2, b2):
    src = edge_index[0]
    dst = edge_index[1]
    dst2 = dst.reshape(NW, EPW)
    x_pad = jnp.zeros((NP, D), jnp.float32).at[:N].set(x)
    z_h = jnp.zeros((RPT, H), jnp.float32)
    z_deg = jnp.zeros((DR * NP,), jnp.float32)

    (deg_parts,) = _sc_deg(dst2, z_deg)
    (agg0,) = _sc_agg(x_pad, src, dst, z_h)
    h1, rdeg = _tc_layer0(x_pad, agg0, deg_parts, W_self0, W_neigh0,
                          b0[None, :])
    (agg1,) = _sc_agg(h1, src, dst, z_h)
    h2 = _tc_layer(h1, agg1, rdeg, W_self1, W_neigh1, b1[None, :], "relu")
    (agg2,) = _sc_agg(h2, src, dst, z_h)
    out = _tc_layer(h2, agg2, rdeg, W_self2, W_neigh2, b2[None, :],
                    "logsoftmax")
    return out[:N]
